# baseline (speedup 1.0000x reference)
import functools

import jax
import jax.numpy as jnp
from jax import lax
from jax.experimental import pallas as pl
from jax.experimental.pallas import tpu as pltpu
from jax.experimental.pallas import tpu_sc as plsc

NPTS = 8192
NCLD = 8
KNB = 20
R = 256
CB = 512
MAXCH = NPTS // CB
PADN = NPTS + CB
NB = NPTS // R
BN_S = float(1.0 / (1.0 + 1e-5) ** 0.5)
F = 64
KPAD = 32


def _knn_proj_body(scal_ref, xpad_ref, slo_ref, shi_ref, wa_ref, wb_ref,
                   b1_ref, idx_ref, p_ref, q_ref, dstore):
    b = pl.program_id(0)
    lo = scal_ref[0, b]
    hi = scal_ref[1, b]
    nch = (hi - lo + CB - 1) // CB

    x_blk = xpad_ref[pl.ds(b * R, R), :]
    sq_r = jnp.sum(x_blk * x_blk, axis=1, keepdims=True)
    seg_lo = slo_ref[...]
    seg_hi = shi_ref[...]

    p_ref[...] = jnp.dot(x_blk, wa_ref[...],
                         preferred_element_type=jnp.float32) + b1_ref[...]
    qmat = jnp.dot(x_blk, wb_ref[...], preferred_element_type=jnp.float32)
    q_ref[...] = jnp.concatenate([qmat, jnp.zeros((R, F), jnp.float32)],
                                 axis=1)

    ones_1 = jnp.ones((1, 1), jnp.float32)
    sq_row = lax.dot_general(ones_1, sq_r, (((1,), (1,)), ((), ())),
                             preferred_element_type=jnp.float32)
    sub = lax.broadcasted_iota(jnp.int32, (CB, 1), 0).astype(jnp.float32)
    inf = jnp.float32(jnp.inf)
    bigf = jnp.float32(2.0 ** 30)

    def fill(j, carry):
        m, im = carry
        cs = lo + j * CB
        xc = xpad_ref[pl.ds(cs, CB), :]
        dg = lax.dot_general(xc, x_blk, (((1,), (1,)), ((), ())),
                             preferred_element_type=jnp.float32)
        sqc = jnp.sum(xc * xc, axis=1, keepdims=True)
        d = sq_row + sqc - 2.0 * dg
        colg = sub + jnp.float32(cs)
        mask = (colg >= seg_lo) & (colg < seg_hi)
        tile = jnp.where(mask, d, inf)
        dstore[j] = tile
        v = jnp.min(tile, axis=0, keepdims=True)
        i = jnp.min(jnp.where(tile == v, colg, bigf), axis=0, keepdims=True)
        take = (v < m) | ((v == m) & (i < im))
        return jnp.where(take, v, m), jnp.where(take, i, im)

    m0 = jnp.full((1, R), inf, jnp.float32)
    i0 = jnp.full((1, R), bigf, jnp.float32)
    m, im = lax.fori_loop(0, nch, fill, (m0, i0))

    kiota = lax.broadcasted_iota(jnp.int32, (KPAD, R), 0)
    acc = jnp.where(kiota == 0, im.astype(jnp.int32), 0)
    for t in range(1, KNB):
        def pick(j, carry):
            m2, im2 = carry
            cs = lo + j * CB
            colg = sub + jnp.float32(cs)
            tile = dstore[j]
            gt = (tile > m) | ((tile == m) & (colg > im))
            cv = jnp.where(gt, tile, inf)
            v = jnp.min(cv, axis=0, keepdims=True)
            i = jnp.min(jnp.where(cv == v, colg, bigf), axis=0,
                        keepdims=True)
            take = (v < m2) | ((v == m2) & (i < im2))
            return jnp.where(take, v, m2), jnp.where(take, i, im2)

        m, im = lax.fori_loop(0, nch, pick, (m0, i0))
        acc = jnp.where(kiota == t, im.astype(jnp.int32), acc)
    idx_ref[...] = acc


def _knn_proj(xpad, seg_lo, seg_hi, scal, wa, wb, b1):
    c_in = xpad.shape[1]
    grid_spec = pltpu.PrefetchScalarGridSpec(
        num_scalar_prefetch=1,
        grid=(NB,),
        in_specs=[
            pl.BlockSpec((PADN, c_in), lambda b, s: (0, 0)),
            pl.BlockSpec((1, R), lambda b, s: (0, b)),
            pl.BlockSpec((1, R), lambda b, s: (0, b)),
            pl.BlockSpec((c_in, F), lambda b, s: (0, 0)),
            pl.BlockSpec((c_in, F), lambda b, s: (0, 0)),
            pl.BlockSpec((1, F), lambda b, s: (0, 0)),
        ],
        out_specs=[
            pl.BlockSpec((KPAD, R), lambda b, s: (0, b)),
            pl.BlockSpec((R, F), lambda b, s: (b, 0)),
            pl.BlockSpec((R, 2 * F), lambda b, s: (b, 0)),
        ],
        scratch_shapes=[pltpu.VMEM((MAXCH, CB, R), jnp.float32)],
    )
    return pl.pallas_call(
        _knn_proj_body,
        grid_spec=grid_spec,
        out_shape=[
            jax.ShapeDtypeStruct((KPAD, NPTS), jnp.int32),
            jax.ShapeDtypeStruct((NPTS, F), jnp.float32),
            jax.ShapeDtypeStruct((NPTS, 2 * F), jnp.float32),
        ],
    )(scal, xpad, seg_lo, seg_hi, wa, wb, b1)


GW = 128


def _sc_gather(table, idx_flat):
    num_idx = idx_flat.shape[0]
    idx2 = idx_flat.reshape(1, num_idx)
    mesh = plsc.VectorSubcoreMesh(core_axis_name="c", subcore_axis_name="s")

    @functools.partial(
        pl.kernel,
        out_type=jax.ShapeDtypeStruct((num_idx, 2 * F), jnp.float32),
        mesh=mesh)
    def kern(x_hbm, i_hbm, o_hbm):
        def body(i_vmem, o_vmem):
            pltpu.sync_copy(x_hbm.at[i_vmem.at[0]], o_vmem)

        pltpu.emit_pipeline(
            body,
            grid=(num_idx // GW,),
            in_specs=[pl.BlockSpec((1, GW), index_map=lambda i: (0, i))],
            out_specs=[pl.BlockSpec((GW, 2 * F), index_map=lambda i: (i, 0))],
            core_axis_name=("c", "s"),
            dimension_semantics=(pltpu.PARALLEL,),
        )(i_hbm, o_hbm)

    return kern(table, idx2)


def _edge_mlp_body(qg_ref, p_ref, w2_ref, b2_ref, o_ref):
    a = jnp.maximum(qg_ref[:, :, :F] + p_ref[...][None], 0.0)
    h = jnp.dot(a.reshape(KNB * R, F), w2_ref[...],
                preferred_element_type=jnp.float32) + b2_ref[...]
    h = jnp.maximum(h, 0.0).reshape(KNB, R, F)
    o_ref[...] = jnp.max(h, axis=0) * BN_S


def _edge_mlp(qg, p, w2s, b2):
    npts = p.shape[0]
    return pl.pallas_call(
        _edge_mlp_body,
        grid=(npts // R,),
        in_specs=[
            pl.BlockSpec((KNB, R, 2 * F), lambda b: (0, b, 0)),
            pl.BlockSpec((R, F), lambda b: (b, 0)),
            pl.BlockSpec((F, F), lambda b: (0, 0)),
            pl.BlockSpec((1, F), lambda b: (0, 0)),
        ],
        out_specs=pl.BlockSpec((R, F), lambda b: (b, 0)),
        out_shape=jax.ShapeDtypeStruct((npts, F), jnp.float32),
    )(qg, p, w2s, b2)


def _head_body(x1_ref, x2_ref, x3_ref, bt_ref, wl_ref, bl_ref, w1_ref,
               c1_ref, w2_ref, c2_ref, w3_ref, c3_ref, o_ref, acc_ref):
    b = pl.program_id(0)

    @pl.when(b == 0)
    def _():
        acc_ref[...] = jnp.full((NCLD, 1024), -jnp.inf, jnp.float32)

    xc = jnp.concatenate([x1_ref[...], x2_ref[...], x3_ref[...]], axis=1)
    o = jnp.maximum(jnp.dot(xc, wl_ref[...],
                            preferred_element_type=jnp.float32)
                    + bl_ref[...], 0.0) * BN_S
    bt = bt_ref[...]
    neg = jnp.float32(-jnp.inf)
    for c in range(NCLD):
        v = jnp.max(jnp.where(bt == c, o, neg), axis=0, keepdims=True)
        acc_ref[pl.ds(c, 1), :] = jnp.maximum(acc_ref[pl.ds(c, 1), :], v)

    @pl.when(b == NB - 1)
    def _():
        h = acc_ref[...]
        h = jnp.maximum(jnp.dot(h, w1_ref[...],
                                preferred_element_type=jnp.float32)
                        + c1_ref[...], 0.0) * BN_S
        h = jnp.maximum(jnp.dot(h, w2_ref[...],
                                preferred_element_type=jnp.float32)
                        + c2_ref[...], 0.0) * BN_S
        o_ref[...] = jnp.dot(h, w3_ref[...],
                             preferred_element_type=jnp.float32) + c3_ref[...]


def _head(x1, x2, x3, bt, wl, bl, w1, c1, w2, c2, w3, c3):
    return pl.pallas_call(
        _head_body,
        grid=(NB,),
        in_specs=[
            pl.BlockSpec((R, F), lambda b: (b, 0)),
            pl.BlockSpec((R, F), lambda b: (b, 0)),
            pl.BlockSpec((R, F), lambda b: (b, 0)),
            pl.BlockSpec((R, 1), lambda b: (b, 0)),
            pl.BlockSpec((192, 1024), lambda b: (0, 0)),
            pl.BlockSpec((1, 1024), lambda b: (0, 0)),
            pl.BlockSpec((1024, 512), lambda b: (0, 0)),
            pl.BlockSpec((1, 512), lambda b: (0, 0)),
            pl.BlockSpec((512, 256), lambda b: (0, 0)),
            pl.BlockSpec((1, 256), lambda b: (0, 0)),
            pl.BlockSpec((256, 128), lambda b: (0, 0)),
            pl.BlockSpec((1, 128), lambda b: (0, 0)),
        ],
        out_specs=pl.BlockSpec((NCLD, 128), lambda b: (0, 0)),
        out_shape=jax.ShapeDtypeStruct((NCLD, 128), jnp.float32),
        scratch_shapes=[pltpu.VMEM((NCLD, 1024), jnp.float32)],
    )(x1, x2, x3, bt, wl, bl, w1, c1, w2, c2, w3, c3)


def _layer(x, seg_lo, seg_hi, scal, blocks):
    c_in = x.shape[1]
    w1 = blocks[0]["W"]
    wa = w1[:c_in] - w1[c_in:]
    wb = w1[c_in:]
    b1 = blocks[0]["b"].reshape(1, F)
    w2s = blocks[1]["W"] * BN_S
    b2 = blocks[1]["b"].reshape(1, F)

    xpad = jnp.zeros((PADN, c_in), jnp.float32).at[:NPTS].set(x)
    idx, p, q = _knn_proj(xpad, seg_lo, seg_hi, scal, wa, wb, b1)
    halves = []
    hp = NPTS // 2
    for h in range(2):
        sl = slice(h * hp, (h + 1) * hp)
        idx_flat = idx[:KNB, sl].reshape(KNB * hp)
        qg = _sc_gather(q, idx_flat).reshape(KNB, hp, 2 * F)
        halves.append(_edge_mlp(qg, p[sl], w2s, b2))
    return jnp.concatenate(halves, axis=0)


def kernel(x, pos, batch, params):
    batch = batch.astype(jnp.int32)
    x0 = jnp.concatenate([x, pos], axis=1)

    bounds = jnp.searchsorted(batch, jnp.arange(NCLD + 1, dtype=jnp.int32),
                              side="left").astype(jnp.int32)
    seg_lo_i = jnp.take(bounds, batch)
    seg_hi_i = jnp.take(bounds, batch + 1)
    r0 = jnp.arange(NB, dtype=jnp.int32) * R
    lo_arr = (seg_lo_i[r0] // 8) * 8
    hi_arr = seg_hi_i[r0 + (R - 1)]
    scal = jnp.stack([lo_arr, hi_arr]).astype(jnp.int32)
    seg_lo = seg_lo_i.astype(jnp.float32).reshape(1, NPTS)
    seg_hi = seg_hi_i.astype(jnp.float32).reshape(1, NPTS)

    x1 = _layer(x0, seg_lo, seg_hi, scal, params["conv1"])
    x2 = _layer(x1, seg_lo, seg_hi, scal, params["conv2"])
    x3 = _layer(x2, seg_lo, seg_hi, scal, params["conv3"])

    bt = batch.reshape(NPTS, 1)
    pl1 = params["lin1"]
    pm1, pm2, pm3 = params["m1"], params["m2"], params["m3"]
    return _head(x1, x2, x3, bt,
                 pl1["W"], pl1["b"].reshape(1, 1024),
                 pm1["W"], pm1["b"].reshape(1, 512),
                 pm2["W"], pm2["b"].reshape(1, 256),
                 pm3["W"], pm3["b"].reshape(1, 128))


# speedup vs baseline: 1.3550x; 1.3550x over previous
import functools

import jax
import jax.numpy as jnp
from jax import lax
from jax.experimental import pallas as pl
from jax.experimental.pallas import tpu as pltpu
from jax.experimental.pallas import tpu_sc as plsc

NPTS = 8192
NCLD = 8
KNB = 20
R = 256
CB = 512
MAXCH = NPTS // CB
PADN = NPTS + CB
NB = NPTS // R
BN_S = float(1.0 / (1.0 + 1e-5) ** 0.5)
F = 64
KPAD = 32


def _knn_proj_body(scal_ref, xpad_ref, slo_ref, shi_ref, wa_ref, wb_ref,
                   b1_ref, idx_ref, p_ref, q_ref, dstore):
    b = pl.program_id(0)
    lo = scal_ref[0, b]
    hi = scal_ref[1, b]
    nch = (hi - lo + CB - 1) // CB

    x_blk = xpad_ref[pl.ds(b * R, R), :]
    sq_r = jnp.sum(x_blk * x_blk, axis=1, keepdims=True)
    seg_lo = slo_ref[...]
    seg_hi = shi_ref[...]

    p_ref[...] = jnp.dot(x_blk, wa_ref[...],
                         preferred_element_type=jnp.float32) + b1_ref[...]
    qmat = jnp.dot(x_blk, wb_ref[...], preferred_element_type=jnp.float32)
    q_ref[...] = jnp.concatenate([qmat, jnp.zeros((R, F), jnp.float32)],
                                 axis=1)

    ones_1 = jnp.ones((1, 1), jnp.float32)
    sq_row = lax.dot_general(ones_1, sq_r, (((1,), (1,)), ((), ())),
                             preferred_element_type=jnp.float32)
    sub = lax.broadcasted_iota(jnp.int32, (CB, 1), 0).astype(jnp.float32)
    inf = jnp.float32(jnp.inf)
    bigf = jnp.float32(2.0 ** 30)

    def fill(j, carry):
        m, im = carry
        cs = lo + j * CB
        xc = xpad_ref[pl.ds(cs, CB), :]
        dg = lax.dot_general(xc, x_blk, (((1,), (1,)), ((), ())),
                             preferred_element_type=jnp.float32)
        sqc = jnp.sum(xc * xc, axis=1, keepdims=True)
        d = sq_row + sqc - 2.0 * dg
        colg = sub + jnp.float32(cs)
        mask = (colg >= seg_lo) & (colg < seg_hi)
        tile = jnp.where(mask, d, inf)
        dstore[j] = tile
        v = jnp.min(tile, axis=0, keepdims=True)
        i = jnp.min(jnp.where(tile == v, colg, bigf), axis=0, keepdims=True)
        take = (v < m) | ((v == m) & (i < im))
        return jnp.where(take, v, m), jnp.where(take, i, im)

    m0 = jnp.full((1, R), inf, jnp.float32)
    i0 = jnp.full((1, R), bigf, jnp.float32)
    m, im = lax.fori_loop(0, nch, fill, (m0, i0))

    kiota = lax.broadcasted_iota(jnp.int32, (KPAD, R), 0)
    acc = jnp.where(kiota == 0, im.astype(jnp.int32), 0)
    for t in range(1, KNB):
        def pick(j, carry):
            m2, im2 = carry
            cs = lo + j * CB
            colg = sub + jnp.float32(cs)
            tile = jnp.where(colg == im, inf, dstore[j])
            if t < KNB - 1:
                dstore[j] = tile
            v = jnp.min(tile, axis=0, keepdims=True)
            i = jnp.min(jnp.where(tile == v, colg, bigf), axis=0,
                        keepdims=True)
            take = (v < m2) | ((v == m2) & (i < im2))
            return jnp.where(take, v, m2), jnp.where(take, i, im2)

        m, im = lax.fori_loop(0, nch, pick, (m0, i0))
        acc = jnp.where(kiota == t, im.astype(jnp.int32), acc)
    idx_ref[...] = acc


def _knn_proj(xpad, seg_lo, seg_hi, scal, wa, wb, b1):
    c_in = xpad.shape[1]
    grid_spec = pltpu.PrefetchScalarGridSpec(
        num_scalar_prefetch=1,
        grid=(NB,),
        in_specs=[
            pl.BlockSpec((PADN, c_in), lambda b, s: (0, 0)),
            pl.BlockSpec((1, R), lambda b, s: (0, b)),
            pl.BlockSpec((1, R), lambda b, s: (0, b)),
            pl.BlockSpec((c_in, F), lambda b, s: (0, 0)),
            pl.BlockSpec((c_in, F), lambda b, s: (0, 0)),
            pl.BlockSpec((1, F), lambda b, s: (0, 0)),
        ],
        out_specs=[
            pl.BlockSpec((KPAD, R), lambda b, s: (0, b)),
            pl.BlockSpec((R, F), lambda b, s: (b, 0)),
            pl.BlockSpec((R, 2 * F), lambda b, s: (b, 0)),
        ],
        scratch_shapes=[pltpu.VMEM((MAXCH, CB, R), jnp.float32)],
    )
    return pl.pallas_call(
        _knn_proj_body,
        grid_spec=grid_spec,
        out_shape=[
            jax.ShapeDtypeStruct((KPAD, NPTS), jnp.int32),
            jax.ShapeDtypeStruct((NPTS, F), jnp.float32),
            jax.ShapeDtypeStruct((NPTS, 2 * F), jnp.float32),
        ],
    )(scal, xpad, seg_lo, seg_hi, wa, wb, b1)


GW = 128


def _sc_gather(table, idx_flat):
    num_idx = idx_flat.shape[0]
    idx2 = idx_flat.reshape(1, num_idx)
    mesh = plsc.VectorSubcoreMesh(core_axis_name="c", subcore_axis_name="s")

    @functools.partial(
        pl.kernel,
        out_type=jax.ShapeDtypeStruct((num_idx, 2 * F), jnp.float32),
        mesh=mesh)
    def kern(x_hbm, i_hbm, o_hbm):
        def body(i_vmem, o_vmem):
            pltpu.sync_copy(x_hbm.at[i_vmem.at[0]], o_vmem)

        pltpu.emit_pipeline(
            body,
            grid=(num_idx // GW,),
            in_specs=[pl.BlockSpec((1, GW), index_map=lambda i: (0, i))],
            out_specs=[pl.BlockSpec((GW, 2 * F), index_map=lambda i: (i, 0))],
            core_axis_name=("c", "s"),
            dimension_semantics=(pltpu.PARALLEL,),
        )(i_hbm, o_hbm)

    return kern(table, idx2)


def _edge_mlp_body(qg_ref, p_ref, w2_ref, b2_ref, o_ref):
    a = jnp.maximum(qg_ref[:, :, :F] + p_ref[...][None], 0.0)
    h = jnp.dot(a.reshape(KNB * R, F), w2_ref[...],
                preferred_element_type=jnp.float32) + b2_ref[...]
    h = jnp.maximum(h, 0.0).reshape(KNB, R, F)
    o_ref[...] = jnp.max(h, axis=0) * BN_S


def _edge_mlp(qg, p, w2s, b2):
    npts = p.shape[0]
    return pl.pallas_call(
        _edge_mlp_body,
        grid=(npts // R,),
        in_specs=[
            pl.BlockSpec((KNB, R, 2 * F), lambda b: (0, b, 0)),
            pl.BlockSpec((R, F), lambda b: (b, 0)),
            pl.BlockSpec((F, F), lambda b: (0, 0)),
            pl.BlockSpec((1, F), lambda b: (0, 0)),
        ],
        out_specs=pl.BlockSpec((R, F), lambda b: (b, 0)),
        out_shape=jax.ShapeDtypeStruct((npts, F), jnp.float32),
    )(qg, p, w2s, b2)


def _head_body(x1_ref, x2_ref, x3_ref, bt_ref, wl_ref, bl_ref, w1_ref,
               c1_ref, w2_ref, c2_ref, w3_ref, c3_ref, o_ref, acc_ref):
    b = pl.program_id(0)

    @pl.when(b == 0)
    def _():
        acc_ref[...] = jnp.full((NCLD, 1024), -jnp.inf, jnp.float32)

    xc = jnp.concatenate([x1_ref[...], x2_ref[...], x3_ref[...]], axis=1)
    o = jnp.maximum(jnp.dot(xc, wl_ref[...],
                            preferred_element_type=jnp.float32)
                    + bl_ref[...], 0.0) * BN_S
    bt = bt_ref[...]
    neg = jnp.float32(-jnp.inf)
    for c in range(NCLD):
        v = jnp.max(jnp.where(bt == c, o, neg), axis=0, keepdims=True)
        acc_ref[pl.ds(c, 1), :] = jnp.maximum(acc_ref[pl.ds(c, 1), :], v)

    @pl.when(b == NB - 1)
    def _():
        h = acc_ref[...]
        h = jnp.maximum(jnp.dot(h, w1_ref[...],
                                preferred_element_type=jnp.float32)
                        + c1_ref[...], 0.0) * BN_S
        h = jnp.maximum(jnp.dot(h, w2_ref[...],
                                preferred_element_type=jnp.float32)
                        + c2_ref[...], 0.0) * BN_S
        o_ref[...] = jnp.dot(h, w3_ref[...],
                             preferred_element_type=jnp.float32) + c3_ref[...]


def _head(x1, x2, x3, bt, wl, bl, w1, c1, w2, c2, w3, c3):
    return pl.pallas_call(
        _head_body,
        grid=(NB,),
        in_specs=[
            pl.BlockSpec((R, F), lambda b: (b, 0)),
            pl.BlockSpec((R, F), lambda b: (b, 0)),
            pl.BlockSpec((R, F), lambda b: (b, 0)),
            pl.BlockSpec((R, 1), lambda b: (b, 0)),
            pl.BlockSpec((192, 1024), lambda b: (0, 0)),
            pl.BlockSpec((1, 1024), lambda b: (0, 0)),
            pl.BlockSpec((1024, 512), lambda b: (0, 0)),
            pl.BlockSpec((1, 512), lambda b: (0, 0)),
            pl.BlockSpec((512, 256), lambda b: (0, 0)),
            pl.BlockSpec((1, 256), lambda b: (0, 0)),
            pl.BlockSpec((256, 128), lambda b: (0, 0)),
            pl.BlockSpec((1, 128), lambda b: (0, 0)),
        ],
        out_specs=pl.BlockSpec((NCLD, 128), lambda b: (0, 0)),
        out_shape=jax.ShapeDtypeStruct((NCLD, 128), jnp.float32),
        scratch_shapes=[pltpu.VMEM((NCLD, 1024), jnp.float32)],
    )(x1, x2, x3, bt, wl, bl, w1, c1, w2, c2, w3, c3)


def _layer(x, seg_lo, seg_hi, scal, blocks):
    c_in = x.shape[1]
    w1 = blocks[0]["W"]
    wa = w1[:c_in] - w1[c_in:]
    wb = w1[c_in:]
    b1 = blocks[0]["b"].reshape(1, F)
    w2s = blocks[1]["W"] * BN_S
    b2 = blocks[1]["b"].reshape(1, F)

    xpad = jnp.zeros((PADN, c_in), jnp.float32).at[:NPTS].set(x)
    idx, p, q = _knn_proj(xpad, seg_lo, seg_hi, scal, wa, wb, b1)
    idx_flat = idx[:KNB].reshape(KNB * NPTS)
    qg = _sc_gather(q, idx_flat).reshape(KNB, NPTS, 2 * F)
    return _edge_mlp(qg, p, w2s, b2)


def kernel(x, pos, batch, params):
    batch = batch.astype(jnp.int32)
    x0 = jnp.concatenate([x, pos], axis=1)

    bounds = jnp.searchsorted(batch, jnp.arange(NCLD + 1, dtype=jnp.int32),
                              side="left").astype(jnp.int32)
    seg_lo_i = jnp.take(bounds, batch)
    seg_hi_i = jnp.take(bounds, batch + 1)
    r0 = jnp.arange(NB, dtype=jnp.int32) * R
    lo_arr = (seg_lo_i[r0] // 8) * 8
    hi_arr = seg_hi_i[r0 + (R - 1)]
    scal = jnp.stack([lo_arr, hi_arr]).astype(jnp.int32)
    seg_lo = seg_lo_i.astype(jnp.float32).reshape(1, NPTS)
    seg_hi = seg_hi_i.astype(jnp.float32).reshape(1, NPTS)

    x1 = _layer(x0, seg_lo, seg_hi, scal, params["conv1"])
    x2 = _layer(x1, seg_lo, seg_hi, scal, params["conv2"])
    x3 = _layer(x2, seg_lo, seg_hi, scal, params["conv3"])

    bt = batch.reshape(NPTS, 1)
    pl1 = params["lin1"]
    pm1, pm2, pm3 = params["m1"], params["m2"], params["m3"]
    return _head(x1, x2, x3, bt,
                 pl1["W"], pl1["b"].reshape(1, 1024),
                 pm1["W"], pm1["b"].reshape(1, 512),
                 pm2["W"], pm2["b"].reshape(1, 256),
                 pm3["W"], pm3["b"].reshape(1, 128))
